# Initial kernel scaffold; baseline (speedup 1.0000x reference)
import functools

import jax
import jax.numpy as jnp
from jax import lax
from jax.experimental import pallas as pl
from jax.experimental.pallas import tpu as pltpu
from jax.experimental.pallas import tpu_sc as plsc

_N = 10000
_E2 = 320000
_EU = 160000
_DH = 128
_DE = 16
_G = 16

_NC = 2
_NS = 16
_NW = _NC * _NS
_CH = 128

_NCH2 = 79
_PW2 = _NCH2 * _CH
_E2P = _NW * _PW2

_NCH1 = 40
_PW1 = _NCH1 * _CH
_E1P = _NW * _PW1

_RPT = 632
_NPAD = _NS * _RPT
_WA = 32

_BR = 1000
_NB = _N // _BR

_mesh = plsc.VectorSubcoreMesh(core_axis_name="c", subcore_axis_name="s")


@functools.partial(
    pl.kernel,
    out_type=jax.ShapeDtypeStruct((_NC * _NPAD, _WA), jnp.float32),
    mesh=_mesh,
    scratch_types=[
        pltpu.VMEM_SHARED((_NPAD, _WA), jnp.float32),
        pltpu.VMEM((_RPT, _WA), jnp.float32),
        pltpu.VMEM((_CH,), jnp.int32),
        pltpu.VMEM((_CH,), jnp.int32),
        pltpu.VMEM((_CH, _WA), jnp.float32),
    ],
)
def _sc_edge_attr_deg(ea_h, de_h, do_h, z_h, out_h, acc_sh, zb, de_v, do_v, ea_v):
    cid = lax.axis_index("c")
    sid = lax.axis_index("s")
    wid = sid * _NC + cid
    r0 = sid * _RPT
    pltpu.sync_copy(z_h, zb)
    pltpu.sync_copy(zb, acc_sh.at[pl.ds(r0, _RPT)])
    plsc.subcore_barrier()

    def step(i, carry):
        base = wid * _PW1 + i * _CH
        pltpu.sync_copy(ea_h.at[pl.ds(base, _CH)], ea_v)
        pltpu.sync_copy(de_h.at[pl.ds(base, _CH)], de_v)
        pltpu.sync_copy(do_h.at[pl.ds(base, _CH)], do_v)
        pltpu.sync_copy(ea_v, acc_sh.at[de_v], add=True)
        pltpu.sync_copy(ea_v, acc_sh.at[do_v], add=True)
        return carry

    lax.fori_loop(0, _NCH1, step, 0)
    plsc.subcore_barrier()
    pltpu.sync_copy(acc_sh.at[pl.ds(r0, _RPT)], zb)
    pltpu.sync_copy(zb, out_h.at[pl.ds(cid * _NPAD + r0, _RPT)])


@functools.partial(
    pl.kernel,
    out_type=jax.ShapeDtypeStruct((_NC * _NPAD, _DH), jnp.float32),
    mesh=_mesh,
    scratch_types=[
        pltpu.VMEM_SHARED((_NPAD, _DH), jnp.float32),
        pltpu.VMEM((_RPT, _DH), jnp.float32),
        pltpu.VMEM((_CH,), jnp.int32),
        pltpu.VMEM((_CH,), jnp.int32),
        pltpu.VMEM((_CH, _DH), jnp.float32),
        pltpu.SemaphoreType.DMA,
    ],
)
def _sc_gather_scatter(table_h, src_h, dst_h, z_h, out_h, acc_sh, zb, src_v,
                       dst_v, rows_v, sem):
    cid = lax.axis_index("c")
    sid = lax.axis_index("s")
    wid = sid * _NC + cid
    r0 = sid * _RPT
    pltpu.sync_copy(z_h, zb)
    pltpu.sync_copy(zb, acc_sh.at[pl.ds(r0, _RPT)])
    plsc.subcore_barrier()

    def step(i, carry):
        base = wid * _PW2 + i * _CH
        pltpu.sync_copy(src_h.at[pl.ds(base, _CH)], src_v)
        pltpu.sync_copy(dst_h.at[pl.ds(base, _CH)], dst_v)
        pltpu.async_copy(table_h.at[src_v], rows_v, sem).wait()
        pltpu.sync_copy(rows_v, acc_sh.at[dst_v], add=True)
        return carry

    lax.fori_loop(0, _NCH2, step, 0)
    plsc.subcore_barrier()
    pltpu.sync_copy(acc_sh.at[pl.ds(r0, _RPT)], zb)
    pltpu.sync_copy(zb, out_h.at[pl.ds(cid * _NPAD + r0, _RPT)])


def _tc_pre_body(x_r, wg_r, acca_r, we_r, wl_r, be_r, bl_r,
                 hs_r, dinv_r, base2_r):
    acc = acca_r[0] + acca_r[1]
    cnt = acc[:, 16:17] + 1.0
    dinv = lax.rsqrt(cnt)
    h = jnp.dot(x_r[...], wg_r[...], preferred_element_type=jnp.float32)
    hs_r[...] = h * dinv
    dinv_r[...] = jnp.broadcast_to(dinv, (_BR, _DH))
    wl = wl_r[...]
    wl_bot = wl[_DH:, :]
    m = jnp.dot(we_r[...], wl_bot, preferred_element_type=jnp.float32)
    cvec = jnp.dot(be_r[...], wl_bot,
                   preferred_element_type=jnp.float32) + bl_r[...]
    svec = jnp.sum(m, axis=0, keepdims=True)
    base2_r[...] = (jnp.dot(acc[:, :_DE], m, preferred_element_type=jnp.float32)
                    + cnt * cvec + svec)


def _tc_pre(x, wg, acca, we, wl, be2, bl2):
    return pl.pallas_call(
        _tc_pre_body,
        grid=(_NB,),
        in_specs=[
            pl.BlockSpec((_BR, _DH), lambda i: (i, 0)),
            pl.BlockSpec((_DH, _DH), lambda i: (0, 0)),
            pl.BlockSpec((_NC, _BR, _WA), lambda i: (0, i, 0)),
            pl.BlockSpec((_DE, _DH), lambda i: (0, 0)),
            pl.BlockSpec((2 * _DH, _DH), lambda i: (0, 0)),
            pl.BlockSpec((1, _DH), lambda i: (0, 0)),
            pl.BlockSpec((1, _DH), lambda i: (0, 0)),
        ],
        out_specs=[
            pl.BlockSpec((_BR, _DH), lambda i: (i, 0)),
            pl.BlockSpec((_BR, _DH), lambda i: (i, 0)),
            pl.BlockSpec((_BR, _DH), lambda i: (i, 0)),
        ],
        out_shape=[jax.ShapeDtypeStruct((_N, _DH), jnp.float32)] * 3,
    )(x, wg, acca, we, wl, be2, bl2)


def _tc_mid_body(acc1_r, hs_r, dinv_r, wl_r, bg_r, t_r):
    pre = dinv_r[...] * (acc1_r[0] + acc1_r[1] + hs_r[...]) + bg_r[...]
    out1 = jnp.maximum(pre, 0.0)
    wl_top = wl_r[...][:_DH, :]
    t_r[...] = jnp.dot(out1, wl_top, preferred_element_type=jnp.float32)


def _tc_mid(acc1, hs, dinvb, wl, bg2):
    return pl.pallas_call(
        _tc_mid_body,
        grid=(_NB,),
        in_specs=[
            pl.BlockSpec((_NC, _BR, _DH), lambda i: (0, i, 0)),
            pl.BlockSpec((_BR, _DH), lambda i: (i, 0)),
            pl.BlockSpec((_BR, _DH), lambda i: (i, 0)),
            pl.BlockSpec((2 * _DH, _DH), lambda i: (0, 0)),
            pl.BlockSpec((1, _DH), lambda i: (0, 0)),
        ],
        out_specs=pl.BlockSpec((_BR, _DH), lambda i: (i, 0)),
        out_shape=jax.ShapeDtypeStruct((_N, _DH), jnp.float32),
    )(acc1, hs, dinvb, wl, bg2)


def _tc_post_body(acc2_r, t_r, base2_r, batch_r, wo_r, bo_r, out_r,
                  seg_acc, cnt_acc):
    i = pl.program_id(0)
    out2 = jnp.maximum(acc2_r[0] + acc2_r[1] + t_r[...] + base2_r[...], 0.0)
    oh = (lax.broadcasted_iota(jnp.int32, (_G, _BR), 0)
          == batch_r[0]).astype(jnp.float32)

    @pl.when(i == 0)
    def _():
        seg_acc[...] = jnp.zeros((_G, _DH), jnp.float32)
        cnt_acc[...] = jnp.zeros((_G, _DH), jnp.float32)

    seg_acc[...] += jnp.dot(oh, out2, preferred_element_type=jnp.float32)
    cnt_acc[...] += jnp.broadcast_to(
        jnp.sum(oh, axis=1, keepdims=True), (_G, _DH))
    pooled = seg_acc[...] / jnp.maximum(cnt_acc[...], 1.0)
    out_r[...] = jnp.dot(pooled, wo_r[...],
                         preferred_element_type=jnp.float32) + bo_r[...]


def _tc_post(acc2, t, base2, batchr, wo, bo2):
    return pl.pallas_call(
        _tc_post_body,
        grid=(_NB,),
        in_specs=[
            pl.BlockSpec((_NC, _BR, _DH), lambda i: (0, i, 0)),
            pl.BlockSpec((_BR, _DH), lambda i: (i, 0)),
            pl.BlockSpec((_BR, _DH), lambda i: (i, 0)),
            pl.BlockSpec((1, 1, _BR), lambda i: (i, 0, 0)),
            pl.BlockSpec((_DH, _DH), lambda i: (0, 0)),
            pl.BlockSpec((1, _DH), lambda i: (0, 0)),
        ],
        out_specs=pl.BlockSpec((_G, _DH), lambda i: (0, 0)),
        out_shape=jax.ShapeDtypeStruct((_G, _DH), jnp.float32),
        scratch_shapes=[
            pltpu.VMEM((_G, _DH), jnp.float32),
            pltpu.VMEM((_G, _DH), jnp.float32),
        ],
    )(acc2, t, base2, batchr, wo, bo2)


def kernel(x, edge_index, edge_attr, batch, Wg, bg, We, be, Wl, bl, Wo, bo):
    src = edge_index[0].astype(jnp.int32)
    dst = edge_index[1].astype(jnp.int32)

    pad2 = _E2P - _E2
    srcp = jnp.concatenate([src, jnp.zeros((pad2,), jnp.int32)])
    dstp = jnp.concatenate([dst, jnp.full((pad2,), _N, jnp.int32)])

    de = dst[0::2]
    do = dst[1::2]
    pad1 = _E1P - _EU
    dep = jnp.concatenate([de, jnp.full((pad1,), _N, jnp.int32)])
    dop = jnp.concatenate([do, jnp.full((pad1,), _N, jnp.int32)])
    eap = jnp.concatenate(
        [edge_attr, jnp.ones((_EU, 1), jnp.float32),
         jnp.zeros((_EU, _WA - _DE - 1), jnp.float32)], axis=1)
    eap = jnp.concatenate([eap, jnp.zeros((pad1, _WA), jnp.float32)], axis=0)

    z128 = jnp.zeros((_RPT, _DH), jnp.float32)
    z32 = jnp.zeros((_RPT, _WA), jnp.float32)

    acca = _sc_edge_attr_deg(eap, dep, dop, z32)
    acca = acca.reshape(_NC, _NPAD, _WA)[:, :_N, :]

    be2 = be.reshape(1, _DH)
    bl2 = bl.reshape(1, _DH)
    bg2 = bg.reshape(1, _DH)
    bo2 = bo.reshape(1, _DH)

    hs, dinvb, base2 = _tc_pre(x, Wg, acca, We, Wl, be2, bl2)

    acc1 = _sc_gather_scatter(hs, srcp, dstp, z128)
    acc1 = acc1.reshape(_NC, _NPAD, _DH)[:, :_N, :]

    t = _tc_mid(acc1, hs, dinvb, Wl, bg2)

    acc2 = _sc_gather_scatter(t, srcp, dstp, z128)
    acc2 = acc2.reshape(_NC, _NPAD, _DH)[:, :_N, :]

    batchr = batch.astype(jnp.int32).reshape(_NB, 1, _BR)
    return _tc_post(acc2, t, base2, batchr, Wo, bo2)


# speedup vs baseline: 7.4427x; 7.4427x over previous
import functools

import jax
import jax.numpy as jnp
from jax import lax
from jax.experimental import pallas as pl
from jax.experimental.pallas import tpu as pltpu
from jax.experimental.pallas import tpu_sc as plsc

_N = 10000
_E2 = 320000
_EU = 160000
_DH = 128
_HD = 64
_DE = 16
_G = 16

_NC = 2
_NS = 16
_NW = _NC * _NS
_CH = 128

_NCH2 = 158
_PW2 = _NCH2 * _CH
_E2P = _NS * _PW2

_NCH1 = 40
_PW1 = _NCH1 * _CH
_E1P = _NW * _PW1

_RPT = 632
_NPAD = _NS * _RPT
_WA = 32

_BR = 1000
_NB = _N // _BR

_mesh = plsc.VectorSubcoreMesh(core_axis_name="c", subcore_axis_name="s")


@functools.partial(
    pl.kernel,
    out_type=jax.ShapeDtypeStruct((_NC * _NPAD, _WA), jnp.float32),
    mesh=_mesh,
    scratch_types=[
        pltpu.VMEM_SHARED((_NPAD, _WA), jnp.float32),
        pltpu.VMEM((_RPT, _WA), jnp.float32),
        pltpu.VMEM((_CH,), jnp.int32),
        pltpu.VMEM((_CH,), jnp.int32),
        pltpu.VMEM((_CH, _WA), jnp.float32),
    ],
    compiler_params=pltpu.CompilerParams(use_tc_tiling_on_sc=False),
)
def _sc_edge_attr_deg(ea_h, de_h, do_h, z_h, out_h, acc_sh, zb, de_v, do_v, ea_v):
    cid = lax.axis_index("c")
    sid = lax.axis_index("s")
    wid = sid * _NC + cid
    r0 = sid * _RPT
    pltpu.sync_copy(z_h, zb)
    pltpu.sync_copy(zb, acc_sh.at[pl.ds(r0, _RPT)])
    plsc.subcore_barrier()

    def step(i, carry):
        base = wid * _PW1 + i * _CH
        pltpu.sync_copy(ea_h.at[pl.ds(base, _CH)], ea_v)
        pltpu.sync_copy(de_h.at[pl.ds(base, _CH)], de_v)
        pltpu.sync_copy(do_h.at[pl.ds(base, _CH)], do_v)
        pltpu.sync_copy(ea_v, acc_sh.at[de_v], add=True)
        pltpu.sync_copy(ea_v, acc_sh.at[do_v], add=True)
        return carry

    lax.fori_loop(0, _NCH1, step, 0)
    plsc.subcore_barrier()
    pltpu.sync_copy(acc_sh.at[pl.ds(r0, _RPT)], zb)
    pltpu.sync_copy(zb, out_h.at[pl.ds(cid * _NPAD + r0, _RPT)])


@functools.partial(
    pl.kernel,
    out_type=jax.ShapeDtypeStruct((_NC * _NPAD, _HD), jnp.float32),
    mesh=_mesh,
    scratch_types=[
        pltpu.VMEM_SHARED((_NPAD, _HD), jnp.float32),
        pltpu.VMEM((_RPT, _HD), jnp.float32),
        pltpu.VMEM((_CH,), jnp.int32),
        pltpu.VMEM((_CH,), jnp.int32),
        pltpu.VMEM((_CH, _HD), jnp.float32),
        pltpu.SemaphoreType.DMA,
    ],
    compiler_params=pltpu.CompilerParams(use_tc_tiling_on_sc=False),
)
def _sc_gather_scatter(table_h, src_h, dst_h, z_h, out_h, acc_sh, zb, src_v,
                       dst_v, rows_v, sem):
    cid = lax.axis_index("c")
    sid = lax.axis_index("s")
    r0 = sid * _RPT
    pltpu.sync_copy(z_h, zb)
    pltpu.sync_copy(zb, acc_sh.at[pl.ds(r0, _RPT)])
    plsc.subcore_barrier()

    def step(i, carry):
        base = sid * _PW2 + i * _CH
        pltpu.sync_copy(src_h.at[pl.ds(cid * _E2P + base, _CH)], src_v)
        pltpu.sync_copy(dst_h.at[pl.ds(base, _CH)], dst_v)
        pltpu.async_copy(table_h.at[src_v], rows_v, sem).wait()
        pltpu.sync_copy(rows_v, acc_sh.at[dst_v], add=True)
        return carry

    lax.fori_loop(0, _NCH2, step, 0)
    plsc.subcore_barrier()
    pltpu.sync_copy(acc_sh.at[pl.ds(r0, _RPT)], zb)
    pltpu.sync_copy(zb, out_h.at[pl.ds(cid * _NPAD + r0, _RPT)])


def _tc_pre_body(x_r, wg_r, acca_r, we_r, wl_r, be_r, bl_r,
                 hs_r, dinv_r, base2_r):
    acc = acca_r[0] + acca_r[1]
    cnt = acc[:, 16:17] + 1.0
    dinv = lax.rsqrt(cnt)
    h = jnp.dot(x_r[...], wg_r[...], preferred_element_type=jnp.float32)
    hs = h * dinv
    hs_r[0] = hs[:, :_HD]
    hs_r[1] = hs[:, _HD:]
    dinv_r[...] = jnp.broadcast_to(dinv, (_BR, _DH))
    wl = wl_r[...]
    wl_bot = wl[_DH:, :]
    m = jnp.dot(we_r[...], wl_bot, preferred_element_type=jnp.float32)
    cvec = jnp.dot(be_r[...], wl_bot,
                   preferred_element_type=jnp.float32) + bl_r[...]
    svec = jnp.sum(m, axis=0, keepdims=True)
    base2_r[...] = (jnp.dot(acc[:, :_DE], m, preferred_element_type=jnp.float32)
                    + cnt * cvec + svec)


def _tc_pre(x, wg, acca, we, wl, be2, bl2):
    return pl.pallas_call(
        _tc_pre_body,
        grid=(_NB,),
        in_specs=[
            pl.BlockSpec((_BR, _DH), lambda i: (i, 0)),
            pl.BlockSpec((_DH, _DH), lambda i: (0, 0)),
            pl.BlockSpec((_NC, _BR, _WA), lambda i: (0, i, 0)),
            pl.BlockSpec((_DE, _DH), lambda i: (0, 0)),
            pl.BlockSpec((2 * _DH, _DH), lambda i: (0, 0)),
            pl.BlockSpec((1, _DH), lambda i: (0, 0)),
            pl.BlockSpec((1, _DH), lambda i: (0, 0)),
        ],
        out_specs=[
            pl.BlockSpec((_NC, _BR, _HD), lambda i: (0, i, 0)),
            pl.BlockSpec((_BR, _DH), lambda i: (i, 0)),
            pl.BlockSpec((_BR, _DH), lambda i: (i, 0)),
        ],
        out_shape=[
            jax.ShapeDtypeStruct((_NC, _N, _HD), jnp.float32),
            jax.ShapeDtypeStruct((_N, _DH), jnp.float32),
            jax.ShapeDtypeStruct((_N, _DH), jnp.float32),
        ],
    )(x, wg, acca, we, wl, be2, bl2)


def _tc_mid_body(acc1_r, hs_r, dinv_r, wl_r, bg_r, t_r):
    acc1 = jnp.concatenate([acc1_r[0], acc1_r[1]], axis=1)
    hs = jnp.concatenate([hs_r[0], hs_r[1]], axis=1)
    pre = dinv_r[...] * (acc1 + hs) + bg_r[...]
    out1 = jnp.maximum(pre, 0.0)
    wl_top = wl_r[...][:_DH, :]
    t = jnp.dot(out1, wl_top, preferred_element_type=jnp.float32)
    t_r[0] = t[:, :_HD]
    t_r[1] = t[:, _HD:]


def _tc_mid(acc1, hs_s, dinvb, wl, bg2):
    return pl.pallas_call(
        _tc_mid_body,
        grid=(_NB,),
        in_specs=[
            pl.BlockSpec((_NC, _BR, _HD), lambda i: (0, i, 0)),
            pl.BlockSpec((_NC, _BR, _HD), lambda i: (0, i, 0)),
            pl.BlockSpec((_BR, _DH), lambda i: (i, 0)),
            pl.BlockSpec((2 * _DH, _DH), lambda i: (0, 0)),
            pl.BlockSpec((1, _DH), lambda i: (0, 0)),
        ],
        out_specs=pl.BlockSpec((_NC, _BR, _HD), lambda i: (0, i, 0)),
        out_shape=jax.ShapeDtypeStruct((_NC, _N, _HD), jnp.float32),
    )(acc1, hs_s, dinvb, wl, bg2)


def _tc_post_body(acc2_r, t_r, base2_r, batch_r, wo_r, bo_r, out_r,
                  seg_acc, cnt_acc):
    i = pl.program_id(0)
    acc2 = jnp.concatenate([acc2_r[0], acc2_r[1]], axis=1)
    t = jnp.concatenate([t_r[0], t_r[1]], axis=1)
    out2 = jnp.maximum(acc2 + t + base2_r[...], 0.0)
    oh = (lax.broadcasted_iota(jnp.int32, (_G, _BR), 0)
          == batch_r[0]).astype(jnp.float32)

    @pl.when(i == 0)
    def _():
        seg_acc[...] = jnp.zeros((_G, _DH), jnp.float32)
        cnt_acc[...] = jnp.zeros((_G, _DH), jnp.float32)

    seg_acc[...] += jnp.dot(oh, out2, preferred_element_type=jnp.float32)
    cnt_acc[...] += jnp.broadcast_to(
        jnp.sum(oh, axis=1, keepdims=True), (_G, _DH))
    pooled = seg_acc[...] / jnp.maximum(cnt_acc[...], 1.0)
    out_r[...] = jnp.dot(pooled, wo_r[...],
                         preferred_element_type=jnp.float32) + bo_r[...]


def _tc_post(acc2, t_s, base2, batchr, wo, bo2):
    return pl.pallas_call(
        _tc_post_body,
        grid=(_NB,),
        in_specs=[
            pl.BlockSpec((_NC, _BR, _HD), lambda i: (0, i, 0)),
            pl.BlockSpec((_NC, _BR, _HD), lambda i: (0, i, 0)),
            pl.BlockSpec((_BR, _DH), lambda i: (i, 0)),
            pl.BlockSpec((1, 1, _BR), lambda i: (i, 0, 0)),
            pl.BlockSpec((_DH, _DH), lambda i: (0, 0)),
            pl.BlockSpec((1, _DH), lambda i: (0, 0)),
        ],
        out_specs=pl.BlockSpec((_G, _DH), lambda i: (0, 0)),
        out_shape=jax.ShapeDtypeStruct((_G, _DH), jnp.float32),
        scratch_shapes=[
            pltpu.VMEM((_G, _DH), jnp.float32),
            pltpu.VMEM((_G, _DH), jnp.float32),
        ],
    )(acc2, t_s, base2, batchr, wo, bo2)


def kernel(x, edge_index, edge_attr, batch, Wg, bg, We, be, Wl, bl, Wo, bo):
    src = edge_index[0].astype(jnp.int32)
    dst = edge_index[1].astype(jnp.int32)

    pad2 = _E2P - _E2
    srcp = jnp.concatenate([src, jnp.zeros((pad2,), jnp.int32)])
    dstp = jnp.concatenate([dst, jnp.full((pad2,), _N, jnp.int32)])
    src2 = jnp.concatenate([srcp, srcp + _N])

    de = dst[0::2]
    do = dst[1::2]
    pad1 = _E1P - _EU
    dep = jnp.concatenate([de, jnp.full((pad1,), _N, jnp.int32)])
    dop = jnp.concatenate([do, jnp.full((pad1,), _N, jnp.int32)])
    eap = jnp.concatenate(
        [edge_attr, jnp.ones((_EU, 1), jnp.float32),
         jnp.zeros((_EU, _WA - _DE - 1), jnp.float32)], axis=1)
    eap = jnp.concatenate([eap, jnp.zeros((pad1, _WA), jnp.float32)], axis=0)

    z64 = jnp.zeros((_RPT, _HD), jnp.float32)
    z32 = jnp.zeros((_RPT, _WA), jnp.float32)

    acca = _sc_edge_attr_deg(eap, dep, dop, z32)
    acca = acca.reshape(_NC, _NPAD, _WA)[:, :_N, :]

    be2 = be.reshape(1, _DH)
    bl2 = bl.reshape(1, _DH)
    bg2 = bg.reshape(1, _DH)
    bo2 = bo.reshape(1, _DH)

    hs_s, dinvb, base2 = _tc_pre(x, Wg, acca, We, Wl, be2, bl2)

    acc1 = _sc_gather_scatter(hs_s.reshape(_NC * _N, _HD), src2, dstp, z64)
    acc1 = acc1.reshape(_NC, _NPAD, _HD)[:, :_N, :]

    t_s = _tc_mid(acc1, hs_s, dinvb, Wl, bg2)

    acc2 = _sc_gather_scatter(t_s.reshape(_NC * _N, _HD), src2, dstp, z64)
    acc2 = acc2.reshape(_NC, _NPAD, _HD)[:, :_N, :]

    batchr = batch.astype(jnp.int32).reshape(_NB, 1, _BR)
    return _tc_post(acc2, t_s, base2, batchr, Wo, bo2)


# speedup vs baseline: 8.6880x; 1.1673x over previous
import functools

import jax
import jax.numpy as jnp
from jax import lax
from jax.experimental import pallas as pl
from jax.experimental.pallas import tpu as pltpu
from jax.experimental.pallas import tpu_sc as plsc

_N = 10000
_E2 = 320000
_EU = 160000
_DH = 128
_HD = 64
_DE = 16
_G = 16

_NC = 2
_NS = 16
_NW = _NC * _NS
_CH = 128

_KB = 4
_SB = _KB * _CH

_S2 = 40
_PW2 = _S2 * _SB
_E2P = _NS * _PW2

_KA = 8
_SA = _KA * _CH
_S1 = 5
_PW1 = _S1 * _SA
_E1P = _NW * _PW1

_RPT = 632
_NPAD = _NS * _RPT
_WA = 32

_BR = 1000
_NB = _N // _BR

_mesh = plsc.VectorSubcoreMesh(core_axis_name="c", subcore_axis_name="s")


@functools.partial(
    pl.kernel,
    out_type=jax.ShapeDtypeStruct((_NC * _NPAD, _WA), jnp.float32),
    mesh=_mesh,
    scratch_types=[
        pltpu.VMEM_SHARED((_NPAD, _WA), jnp.float32),
        pltpu.VMEM((_KA, _CH), jnp.int32),
        pltpu.VMEM((_KA, _CH), jnp.int32),
        pltpu.VMEM((_SA, _WA), jnp.float32),
        pltpu.SemaphoreType.DMA,
    ],
    compiler_params=pltpu.CompilerParams(use_tc_tiling_on_sc=False),
)
def _sc_edge_attr_deg(ea_h, de_h, do_h, z_h, out_h, acc_sh, de_v, do_v,
                      ea_v, sem):
    cid = lax.axis_index("c")
    sid = lax.axis_index("s")
    wid = sid * _NC + cid
    r0 = sid * _RPT
    pltpu.sync_copy(z_h, acc_sh.at[pl.ds(r0, _RPT)])
    plsc.subcore_barrier()

    def step(g, carry):
        base = wid * _PW1 + g * _SA
        rb = wid * (_PW1 // _CH) + g * _KA
        pltpu.sync_copy(ea_h.at[pl.ds(base, _SA)], ea_v)
        pltpu.sync_copy(de_h.at[pl.ds(rb, _KA)], de_v)
        pltpu.sync_copy(do_h.at[pl.ds(rb, _KA)], do_v)
        descs = (
            [pltpu.async_copy(ea_v.at[pl.ds(j * _CH, _CH)],
                              acc_sh.at[de_v.at[j]], sem, add=True)
             for j in range(_KA)]
            + [pltpu.async_copy(ea_v.at[pl.ds(j * _CH, _CH)],
                                acc_sh.at[do_v.at[j]], sem, add=True)
               for j in range(_KA)])
        for d in descs:
            d.wait()
        return carry

    lax.fori_loop(0, _S1, step, 0)
    plsc.subcore_barrier()
    pltpu.sync_copy(acc_sh.at[pl.ds(r0, _RPT)],
                    out_h.at[pl.ds(cid * _NPAD + r0, _RPT)])


@functools.partial(
    pl.kernel,
    out_type=jax.ShapeDtypeStruct((_NC * _NPAD, _HD), jnp.float32),
    mesh=_mesh,
    scratch_types=[
        pltpu.VMEM_SHARED((_NPAD, _HD), jnp.float32),
        pltpu.VMEM((_KB, _CH), jnp.int32),
        pltpu.VMEM((_KB, _CH), jnp.int32),
        pltpu.VMEM((_KB, _CH, _HD), jnp.float32),
        pltpu.SemaphoreType.DMA,
        pltpu.SemaphoreType.DMA,
    ],
    compiler_params=pltpu.CompilerParams(use_tc_tiling_on_sc=False),
)
def _sc_gather_scatter(table_h, src_h, dst_h, z_h, out_h, acc_sh, src_v,
                       dst_v, rows_v, gsem, ssem):
    cid = lax.axis_index("c")
    sid = lax.axis_index("s")
    r0 = sid * _RPT
    pltpu.sync_copy(z_h, acc_sh.at[pl.ds(r0, _RPT)])
    plsc.subcore_barrier()

    def step(g, carry):
        rb = sid * (_PW2 // _CH) + g * _KB
        pltpu.sync_copy(src_h.at[pl.ds(cid * (_E2P // _CH) + rb, _KB)], src_v)
        pltpu.sync_copy(dst_h.at[pl.ds(rb, _KB)], dst_v)
        gd = [pltpu.async_copy(table_h.at[src_v.at[j]], rows_v.at[j], gsem)
              for j in range(_KB)]
        for d in gd:
            d.wait()
        sd = [pltpu.async_copy(rows_v.at[j], acc_sh.at[dst_v.at[j]], ssem,
                               add=True)
              for j in range(_KB)]
        for d in sd:
            d.wait()
        return carry

    lax.fori_loop(0, _S2, step, 0)
    plsc.subcore_barrier()
    pltpu.sync_copy(acc_sh.at[pl.ds(r0, _RPT)],
                    out_h.at[pl.ds(cid * _NPAD + r0, _RPT)])


def _tc_pre_body(x_r, wg_r, acca_r, we_r, wl_r, be_r, bl_r,
                 hs_r, dinv_r, base2_r):
    acc = acca_r[0] + acca_r[1]
    cnt = acc[:, 16:17] + 1.0
    dinv = lax.rsqrt(cnt)
    h = jnp.dot(x_r[...], wg_r[...], preferred_element_type=jnp.float32)
    hs = h * dinv
    hs_r[0] = hs[:, :_HD]
    hs_r[1] = hs[:, _HD:]
    dinv_r[...] = jnp.broadcast_to(dinv, (_BR, _DH))
    wl = wl_r[...]
    wl_bot = wl[_DH:, :]
    m = jnp.dot(we_r[...], wl_bot, preferred_element_type=jnp.float32)
    cvec = jnp.dot(be_r[...], wl_bot,
                   preferred_element_type=jnp.float32) + bl_r[...]
    svec = jnp.sum(m, axis=0, keepdims=True)
    base2_r[...] = (jnp.dot(acc[:, :_DE], m, preferred_element_type=jnp.float32)
                    + cnt * cvec + svec)


def _tc_pre(x, wg, acca, we, wl, be2, bl2):
    return pl.pallas_call(
        _tc_pre_body,
        grid=(_NB,),
        in_specs=[
            pl.BlockSpec((_BR, _DH), lambda i: (i, 0)),
            pl.BlockSpec((_DH, _DH), lambda i: (0, 0)),
            pl.BlockSpec((_NC, _BR, _WA), lambda i: (0, i, 0)),
            pl.BlockSpec((_DE, _DH), lambda i: (0, 0)),
            pl.BlockSpec((2 * _DH, _DH), lambda i: (0, 0)),
            pl.BlockSpec((1, _DH), lambda i: (0, 0)),
            pl.BlockSpec((1, _DH), lambda i: (0, 0)),
        ],
        out_specs=[
            pl.BlockSpec((_NC, _BR, _HD), lambda i: (0, i, 0)),
            pl.BlockSpec((_BR, _DH), lambda i: (i, 0)),
            pl.BlockSpec((_BR, _DH), lambda i: (i, 0)),
        ],
        out_shape=[
            jax.ShapeDtypeStruct((_NC, _N, _HD), jnp.float32),
            jax.ShapeDtypeStruct((_N, _DH), jnp.float32),
            jax.ShapeDtypeStruct((_N, _DH), jnp.float32),
        ],
    )(x, wg, acca, we, wl, be2, bl2)


def _tc_mid_body(acc1_r, hs_r, dinv_r, wl_r, bg_r, t_r):
    acc1 = jnp.concatenate([acc1_r[0], acc1_r[1]], axis=1)
    hs = jnp.concatenate([hs_r[0], hs_r[1]], axis=1)
    pre = dinv_r[...] * (acc1 + hs) + bg_r[...]
    out1 = jnp.maximum(pre, 0.0)
    wl_top = wl_r[...][:_DH, :]
    t = jnp.dot(out1, wl_top, preferred_element_type=jnp.float32)
    t_r[0] = t[:, :_HD]
    t_r[1] = t[:, _HD:]


def _tc_mid(acc1, hs_s, dinvb, wl, bg2):
    return pl.pallas_call(
        _tc_mid_body,
        grid=(_NB,),
        in_specs=[
            pl.BlockSpec((_NC, _BR, _HD), lambda i: (0, i, 0)),
            pl.BlockSpec((_NC, _BR, _HD), lambda i: (0, i, 0)),
            pl.BlockSpec((_BR, _DH), lambda i: (i, 0)),
            pl.BlockSpec((2 * _DH, _DH), lambda i: (0, 0)),
            pl.BlockSpec((1, _DH), lambda i: (0, 0)),
        ],
        out_specs=pl.BlockSpec((_NC, _BR, _HD), lambda i: (0, i, 0)),
        out_shape=jax.ShapeDtypeStruct((_NC, _N, _HD), jnp.float32),
    )(acc1, hs_s, dinvb, wl, bg2)


def _tc_post_body(acc2_r, t_r, base2_r, batch_r, wo_r, bo_r, out_r,
                  seg_acc, cnt_acc):
    i = pl.program_id(0)
    acc2 = jnp.concatenate([acc2_r[0], acc2_r[1]], axis=1)
    t = jnp.concatenate([t_r[0], t_r[1]], axis=1)
    out2 = jnp.maximum(acc2 + t + base2_r[...], 0.0)
    oh = (lax.broadcasted_iota(jnp.int32, (_G, _BR), 0)
          == batch_r[0]).astype(jnp.float32)

    @pl.when(i == 0)
    def _():
        seg_acc[...] = jnp.zeros((_G, _DH), jnp.float32)
        cnt_acc[...] = jnp.zeros((_G, _DH), jnp.float32)

    seg_acc[...] += jnp.dot(oh, out2, preferred_element_type=jnp.float32)
    cnt_acc[...] += jnp.broadcast_to(
        jnp.sum(oh, axis=1, keepdims=True), (_G, _DH))
    pooled = seg_acc[...] / jnp.maximum(cnt_acc[...], 1.0)
    out_r[...] = jnp.dot(pooled, wo_r[...],
                         preferred_element_type=jnp.float32) + bo_r[...]


def _tc_post(acc2, t_s, base2, batchr, wo, bo2):
    return pl.pallas_call(
        _tc_post_body,
        grid=(_NB,),
        in_specs=[
            pl.BlockSpec((_NC, _BR, _HD), lambda i: (0, i, 0)),
            pl.BlockSpec((_NC, _BR, _HD), lambda i: (0, i, 0)),
            pl.BlockSpec((_BR, _DH), lambda i: (i, 0)),
            pl.BlockSpec((1, 1, _BR), lambda i: (i, 0, 0)),
            pl.BlockSpec((_DH, _DH), lambda i: (0, 0)),
            pl.BlockSpec((1, _DH), lambda i: (0, 0)),
        ],
        out_specs=pl.BlockSpec((_G, _DH), lambda i: (0, 0)),
        out_shape=jax.ShapeDtypeStruct((_G, _DH), jnp.float32),
        scratch_shapes=[
            pltpu.VMEM((_G, _DH), jnp.float32),
            pltpu.VMEM((_G, _DH), jnp.float32),
        ],
    )(acc2, t_s, base2, batchr, wo, bo2)


def kernel(x, edge_index, edge_attr, batch, Wg, bg, We, be, Wl, bl, Wo, bo):
    src = edge_index[0].astype(jnp.int32)
    dst = edge_index[1].astype(jnp.int32)

    pad2 = _E2P - _E2
    srcp = jnp.concatenate([src, jnp.zeros((pad2,), jnp.int32)])
    dstp = jnp.concatenate([dst, jnp.full((pad2,), _N, jnp.int32)])
    src2 = jnp.concatenate([srcp, srcp + _N]).reshape(2 * _E2P // _CH, _CH)
    dstm = dstp.reshape(_E2P // _CH, _CH)

    de = dst[0::2]
    do = dst[1::2]
    pad1 = _E1P - _EU
    dep = jnp.concatenate(
        [de, jnp.full((pad1,), _N, jnp.int32)]).reshape(_E1P // _CH, _CH)
    dop = jnp.concatenate(
        [do, jnp.full((pad1,), _N, jnp.int32)]).reshape(_E1P // _CH, _CH)
    eap = jnp.concatenate(
        [edge_attr, jnp.ones((_EU, 1), jnp.float32),
         jnp.zeros((_EU, _WA - _DE - 1), jnp.float32)], axis=1)
    eap = jnp.concatenate([eap, jnp.zeros((pad1, _WA), jnp.float32)], axis=0)

    z64 = jnp.zeros((_RPT, _HD), jnp.float32)
    z32 = jnp.zeros((_RPT, _WA), jnp.float32)

    acca = _sc_edge_attr_deg(eap, dep, dop, z32)
    acca = acca.reshape(_NC, _NPAD, _WA)[:, :_N, :]

    be2 = be.reshape(1, _DH)
    bl2 = bl.reshape(1, _DH)
    bg2 = bg.reshape(1, _DH)
    bo2 = bo.reshape(1, _DH)

    hs_s, dinvb, base2 = _tc_pre(x, Wg, acca, We, Wl, be2, bl2)

    acc1 = _sc_gather_scatter(hs_s.reshape(_NC * _N, _HD), src2, dstm, z64)
    acc1 = acc1.reshape(_NC, _NPAD, _HD)[:, :_N, :]

    t_s = _tc_mid(acc1, hs_s, dinvb, Wl, bg2)

    acc2 = _sc_gather_scatter(t_s.reshape(_NC * _N, _HD), src2, dstm, z64)
    acc2 = acc2.reshape(_NC, _NPAD, _HD)[:, :_N, :]

    batchr = batch.astype(jnp.int32).reshape(_NB, 1, _BR)
    return _tc_post(acc2, t_s, base2, batchr, Wo, bo2)


# speedup vs baseline: 9.1211x; 1.0498x over previous
import functools

import jax
import jax.numpy as jnp
from jax import lax
from jax.experimental import pallas as pl
from jax.experimental.pallas import tpu as pltpu
from jax.experimental.pallas import tpu_sc as plsc

_N = 10000
_E2 = 320000
_EU = 160000
_DH = 128
_HD = 64
_DE = 16
_G = 16

_NC = 2
_NS = 16
_NW = _NC * _NS
_CH = 128

_KB = 2
_SB = _KB * _CH

_S2 = 80
_PW2 = _S2 * _SB
_E2P = _NS * _PW2

_KA = 8
_SA = _KA * _CH
_S1 = 5
_PW1 = _S1 * _SA
_E1P = _NW * _PW1

_RPT = 632
_NPAD = _NS * _RPT
_WA = 32

_BR = 1000
_NB = _N // _BR

_mesh = plsc.VectorSubcoreMesh(core_axis_name="c", subcore_axis_name="s")


@functools.partial(
    pl.kernel,
    out_type=jax.ShapeDtypeStruct((_NC * _NPAD, _WA), jnp.float32),
    mesh=_mesh,
    scratch_types=[
        pltpu.VMEM_SHARED((_NPAD, _WA), jnp.float32),
        pltpu.VMEM((_KA, _CH), jnp.int32),
        pltpu.VMEM((_KA, _CH), jnp.int32),
        pltpu.VMEM((_SA, _WA), jnp.float32),
        pltpu.SemaphoreType.DMA,
    ],
    compiler_params=pltpu.CompilerParams(use_tc_tiling_on_sc=False),
)
def _sc_edge_attr_deg(ea_h, de_h, do_h, z_h, out_h, acc_sh, de_v, do_v,
                      ea_v, sem):
    cid = lax.axis_index("c")
    sid = lax.axis_index("s")
    wid = sid * _NC + cid
    r0 = sid * _RPT
    pltpu.sync_copy(z_h, acc_sh.at[pl.ds(r0, _RPT)])
    plsc.subcore_barrier()

    def step(g, carry):
        base = wid * _PW1 + g * _SA
        rb = wid * (_PW1 // _CH) + g * _KA
        pltpu.sync_copy(ea_h.at[pl.ds(base, _SA)], ea_v)
        pltpu.sync_copy(de_h.at[pl.ds(rb, _KA)], de_v)
        pltpu.sync_copy(do_h.at[pl.ds(rb, _KA)], do_v)
        descs = (
            [pltpu.async_copy(ea_v.at[pl.ds(j * _CH, _CH)],
                              acc_sh.at[de_v.at[j]], sem, add=True)
             for j in range(_KA)]
            + [pltpu.async_copy(ea_v.at[pl.ds(j * _CH, _CH)],
                                acc_sh.at[do_v.at[j]], sem, add=True)
               for j in range(_KA)])
        for d in descs:
            d.wait()
        return carry

    lax.fori_loop(0, _S1, step, 0)
    plsc.subcore_barrier()
    pltpu.sync_copy(acc_sh.at[pl.ds(r0, _RPT)],
                    out_h.at[pl.ds(cid * _NPAD + r0, _RPT)])


@functools.partial(
    pl.kernel,
    out_type=jax.ShapeDtypeStruct((_NC * _NPAD, _HD), jnp.float32),
    mesh=_mesh,
    scratch_types=[
        pltpu.VMEM_SHARED((_NPAD, _HD), jnp.float32),
        pltpu.VMEM((_KB, _CH), jnp.int32),
        pltpu.VMEM((_KB, _CH), jnp.int32),
        pltpu.VMEM((_KB, _CH), jnp.int32),
        pltpu.VMEM((_KB, _CH), jnp.int32),
        pltpu.VMEM((_KB, _CH, _HD), jnp.float32),
        pltpu.VMEM((_KB, _CH, _HD), jnp.float32),
        pltpu.SemaphoreType.DMA,
        pltpu.SemaphoreType.DMA,
        pltpu.SemaphoreType.DMA,
    ],
    compiler_params=pltpu.CompilerParams(use_tc_tiling_on_sc=False),
)
def _sc_gather_scatter(table_h, src_h, dst_h, z_h, out_h, acc_sh,
                       src_a, dst_a, src_b, dst_b, rows_a, rows_b,
                       gsem, ssem, isem):
    cid = lax.axis_index("c")
    sid = lax.axis_index("s")
    r0 = sid * _RPT
    pltpu.sync_copy(z_h, acc_sh.at[pl.ds(r0, _RPT)])
    plsc.subcore_barrier()

    cbase = cid * (_E2P // _CH)
    tbase = sid * (_PW2 // _CH)

    def load_idx(g, sv, dv):
        rb = tbase + g * _KB
        da = pltpu.async_copy(src_h.at[pl.ds(cbase + rb, _KB)], sv, isem)
        db = pltpu.async_copy(dst_h.at[pl.ds(rb, _KB)], dv, isem)
        return da, db

    def fire_gathers(sv, rows):
        for j in range(_KB):
            pltpu.async_copy(table_h.at[sv.at[j]], rows.at[j], gsem)

    def drain_gathers(sv, rows):
        for j in range(_KB):
            pltpu.make_async_copy(table_h.at[sv.at[j]], rows.at[j],
                                  gsem).wait()

    def fire_scatters(dv, rows):
        for j in range(_KB):
            pltpu.async_copy(rows.at[j], acc_sh.at[dv.at[j]], ssem, add=True)

    def drain_scatters(dv, rows):
        for j in range(_KB):
            pltpu.make_async_copy(rows.at[j], acc_sh.at[dv.at[j]],
                                  ssem).wait()

    da, db = load_idx(0, src_a, dst_a)
    da.wait()
    db.wait()
    fire_gathers(src_a, rows_a)

    def step(i, carry):
        drain_gathers(src_a, rows_a)

        @pl.when(i > 0)
        def _():
            drain_scatters(dst_b, rows_b)

        da, db = load_idx(2 * i + 1, src_b, dst_b)
        fire_scatters(dst_a, rows_a)
        da.wait()
        db.wait()
        fire_gathers(src_b, rows_b)

        drain_gathers(src_b, rows_b)
        drain_scatters(dst_a, rows_a)

        @pl.when(i < _S2 // 2 - 1)
        def _():
            da2, db2 = load_idx(2 * i + 2, src_a, dst_a)
            fire_scatters(dst_b, rows_b)
            da2.wait()
            db2.wait()
            fire_gathers(src_a, rows_a)

        @pl.when(i == _S2 // 2 - 1)
        def _():
            fire_scatters(dst_b, rows_b)

        return carry

    lax.fori_loop(0, _S2 // 2, step, 0)
    drain_scatters(dst_b, rows_b)
    plsc.subcore_barrier()
    pltpu.sync_copy(acc_sh.at[pl.ds(r0, _RPT)],
                    out_h.at[pl.ds(cid * _NPAD + r0, _RPT)])


def _tc_pre_body(x_r, wg_r, acca_r, we_r, wl_r, be_r, bl_r,
                 hs_r, dinv_r, base2_r):
    acc = acca_r[0] + acca_r[1]
    cnt = acc[:, 16:17] + 1.0
    dinv = lax.rsqrt(cnt)
    h = jnp.dot(x_r[...], wg_r[...], preferred_element_type=jnp.float32)
    hs = h * dinv
    hs_r[0] = hs[:, :_HD]
    hs_r[1] = hs[:, _HD:]
    dinv_r[...] = jnp.broadcast_to(dinv, (_BR, _DH))
    wl = wl_r[...]
    wl_bot = wl[_DH:, :]
    m = jnp.dot(we_r[...], wl_bot, preferred_element_type=jnp.float32)
    cvec = jnp.dot(be_r[...], wl_bot,
                   preferred_element_type=jnp.float32) + bl_r[...]
    svec = jnp.sum(m, axis=0, keepdims=True)
    base2_r[...] = (jnp.dot(acc[:, :_DE], m, preferred_element_type=jnp.float32)
                    + cnt * cvec + svec)


def _tc_pre(x, wg, acca, we, wl, be2, bl2):
    return pl.pallas_call(
        _tc_pre_body,
        grid=(_NB,),
        in_specs=[
            pl.BlockSpec((_BR, _DH), lambda i: (i, 0)),
            pl.BlockSpec((_DH, _DH), lambda i: (0, 0)),
            pl.BlockSpec((_NC, _BR, _WA), lambda i: (0, i, 0)),
            pl.BlockSpec((_DE, _DH), lambda i: (0, 0)),
            pl.BlockSpec((2 * _DH, _DH), lambda i: (0, 0)),
            pl.BlockSpec((1, _DH), lambda i: (0, 0)),
            pl.BlockSpec((1, _DH), lambda i: (0, 0)),
        ],
        out_specs=[
            pl.BlockSpec((_NC, _BR, _HD), lambda i: (0, i, 0)),
            pl.BlockSpec((_BR, _DH), lambda i: (i, 0)),
            pl.BlockSpec((_BR, _DH), lambda i: (i, 0)),
        ],
        out_shape=[
            jax.ShapeDtypeStruct((_NC, _N, _HD), jnp.float32),
            jax.ShapeDtypeStruct((_N, _DH), jnp.float32),
            jax.ShapeDtypeStruct((_N, _DH), jnp.float32),
        ],
    )(x, wg, acca, we, wl, be2, bl2)


def _tc_mid_body(acc1_r, hs_r, dinv_r, wl_r, bg_r, t_r):
    acc1 = jnp.concatenate([acc1_r[0], acc1_r[1]], axis=1)
    hs = jnp.concatenate([hs_r[0], hs_r[1]], axis=1)
    pre = dinv_r[...] * (acc1 + hs) + bg_r[...]
    out1 = jnp.maximum(pre, 0.0)
    wl_top = wl_r[...][:_DH, :]
    t = jnp.dot(out1, wl_top, preferred_element_type=jnp.float32)
    t_r[0] = t[:, :_HD]
    t_r[1] = t[:, _HD:]


def _tc_mid(acc1, hs_s, dinvb, wl, bg2):
    return pl.pallas_call(
        _tc_mid_body,
        grid=(_NB,),
        in_specs=[
            pl.BlockSpec((_NC, _BR, _HD), lambda i: (0, i, 0)),
            pl.BlockSpec((_NC, _BR, _HD), lambda i: (0, i, 0)),
            pl.BlockSpec((_BR, _DH), lambda i: (i, 0)),
            pl.BlockSpec((2 * _DH, _DH), lambda i: (0, 0)),
            pl.BlockSpec((1, _DH), lambda i: (0, 0)),
        ],
        out_specs=pl.BlockSpec((_NC, _BR, _HD), lambda i: (0, i, 0)),
        out_shape=jax.ShapeDtypeStruct((_NC, _N, _HD), jnp.float32),
    )(acc1, hs_s, dinvb, wl, bg2)


def _tc_post_body(acc2_r, t_r, base2_r, batch_r, wo_r, bo_r, out_r,
                  seg_acc, cnt_acc):
    i = pl.program_id(0)
    acc2 = jnp.concatenate([acc2_r[0], acc2_r[1]], axis=1)
    t = jnp.concatenate([t_r[0], t_r[1]], axis=1)
    out2 = jnp.maximum(acc2 + t + base2_r[...], 0.0)
    oh = (lax.broadcasted_iota(jnp.int32, (_G, _BR), 0)
          == batch_r[0]).astype(jnp.float32)

    @pl.when(i == 0)
    def _():
        seg_acc[...] = jnp.zeros((_G, _DH), jnp.float32)
        cnt_acc[...] = jnp.zeros((_G, _DH), jnp.float32)

    seg_acc[...] += jnp.dot(oh, out2, preferred_element_type=jnp.float32)
    cnt_acc[...] += jnp.broadcast_to(
        jnp.sum(oh, axis=1, keepdims=True), (_G, _DH))
    pooled = seg_acc[...] / jnp.maximum(cnt_acc[...], 1.0)
    out_r[...] = jnp.dot(pooled, wo_r[...],
                         preferred_element_type=jnp.float32) + bo_r[...]


def _tc_post(acc2, t_s, base2, batchr, wo, bo2):
    return pl.pallas_call(
        _tc_post_body,
        grid=(_NB,),
        in_specs=[
            pl.BlockSpec((_NC, _BR, _HD), lambda i: (0, i, 0)),
            pl.BlockSpec((_NC, _BR, _HD), lambda i: (0, i, 0)),
            pl.BlockSpec((_BR, _DH), lambda i: (i, 0)),
            pl.BlockSpec((1, 1, _BR), lambda i: (i, 0, 0)),
            pl.BlockSpec((_DH, _DH), lambda i: (0, 0)),
            pl.BlockSpec((1, _DH), lambda i: (0, 0)),
        ],
        out_specs=pl.BlockSpec((_G, _DH), lambda i: (0, 0)),
        out_shape=jax.ShapeDtypeStruct((_G, _DH), jnp.float32),
        scratch_shapes=[
            pltpu.VMEM((_G, _DH), jnp.float32),
            pltpu.VMEM((_G, _DH), jnp.float32),
        ],
    )(acc2, t_s, base2, batchr, wo, bo2)


def kernel(x, edge_index, edge_attr, batch, Wg, bg, We, be, Wl, bl, Wo, bo):
    src = edge_index[0].astype(jnp.int32)
    dst = edge_index[1].astype(jnp.int32)

    pad2 = _E2P - _E2
    srcp = jnp.concatenate([src, jnp.zeros((pad2,), jnp.int32)])
    dstp = jnp.concatenate([dst, jnp.full((pad2,), _N, jnp.int32)])
    src2 = jnp.concatenate([srcp, srcp + _N]).reshape(2 * _E2P // _CH, _CH)
    dstm = dstp.reshape(_E2P // _CH, _CH)

    de = dst[0::2]
    do = dst[1::2]
    pad1 = _E1P - _EU
    dep = jnp.concatenate(
        [de, jnp.full((pad1,), _N, jnp.int32)]).reshape(_E1P // _CH, _CH)
    dop = jnp.concatenate(
        [do, jnp.full((pad1,), _N, jnp.int32)]).reshape(_E1P // _CH, _CH)
    eap = jnp.concatenate(
        [edge_attr, jnp.ones((_EU, 1), jnp.float32),
         jnp.zeros((_EU, _WA - _DE - 1), jnp.float32)], axis=1)
    eap = jnp.concatenate([eap, jnp.zeros((pad1, _WA), jnp.float32)], axis=0)

    z64 = jnp.zeros((_RPT, _HD), jnp.float32)
    z32 = jnp.zeros((_RPT, _WA), jnp.float32)

    acca = _sc_edge_attr_deg(eap, dep, dop, z32)
    acca = acca.reshape(_NC, _NPAD, _WA)[:, :_N, :]

    be2 = be.reshape(1, _DH)
    bl2 = bl.reshape(1, _DH)
    bg2 = bg.reshape(1, _DH)
    bo2 = bo.reshape(1, _DH)

    hs_s, dinvb, base2 = _tc_pre(x, Wg, acca, We, Wl, be2, bl2)

    acc1 = _sc_gather_scatter(hs_s.reshape(_NC * _N, _HD), src2, dstm, z64)
    acc1 = acc1.reshape(_NC, _NPAD, _HD)[:, :_N, :]

    t_s = _tc_mid(acc1, hs_s, dinvb, Wl, bg2)

    acc2 = _sc_gather_scatter(t_s.reshape(_NC * _N, _HD), src2, dstm, z64)
    acc2 = acc2.reshape(_NC, _NPAD, _HD)[:, :_N, :]

    batchr = batch.astype(jnp.int32).reshape(_NB, 1, _BR)
    return _tc_post(acc2, t_s, base2, batchr, Wo, bo2)


# speedup vs baseline: 13.4520x; 1.4748x over previous
import functools

import jax
import jax.numpy as jnp
from jax import lax
from jax.experimental import pallas as pl
from jax.experimental.pallas import tpu as pltpu
from jax.experimental.pallas import tpu_sc as plsc

_N = 10000
_E2 = 320000
_EU = 160000
_DH = 128
_HD = 64
_DE = 16
_G = 16

_NC = 2
_NS = 16
_NW = _NC * _NS
_CH = 128

_KB = 4
_SB = _KB * _CH

_S2 = 40
_PW2 = _S2 * _SB
_E2P = _NS * _PW2

_KA = 8
_SA = _KA * _CH
_S1 = 5
_PW1 = _S1 * _SA
_E1P = _NW * _PW1

_RPT = 632
_NPAD = _NS * _RPT
_WA = 32

_BR = 1000
_NB = _N // _BR

_mesh = plsc.VectorSubcoreMesh(core_axis_name="c", subcore_axis_name="s")


@functools.partial(
    pl.kernel,
    out_type=jax.ShapeDtypeStruct((_NC * _NPAD, _WA), jnp.float32),
    mesh=_mesh,
    scratch_types=[
        pltpu.VMEM_SHARED((_NPAD, _WA), jnp.float32),
        pltpu.VMEM((_KA, _CH), jnp.int32),
        pltpu.VMEM((_KA, _CH), jnp.int32),
        pltpu.VMEM((_SA, _WA), jnp.float32),
        pltpu.SemaphoreType.DMA,
    ],
    compiler_params=pltpu.CompilerParams(use_tc_tiling_on_sc=False),
)
def _sc_edge_attr_deg(ea_h, de_h, do_h, z_h, out_h, acc_sh, de_v, do_v,
                      ea_v, sem):
    cid = lax.axis_index("c")
    sid = lax.axis_index("s")
    wid = sid * _NC + cid
    r0 = sid * _RPT
    pltpu.sync_copy(z_h, acc_sh.at[pl.ds(r0, _RPT)])
    plsc.subcore_barrier()

    def step(g, carry):
        base = wid * _PW1 + g * _SA
        rb = wid * (_PW1 // _CH) + g * _KA
        pltpu.sync_copy(ea_h.at[pl.ds(base, _SA)], ea_v)
        pltpu.sync_copy(de_h.at[pl.ds(rb, _KA)], de_v)
        pltpu.sync_copy(do_h.at[pl.ds(rb, _KA)], do_v)
        descs = (
            [pltpu.async_copy(ea_v.at[pl.ds(j * _CH, _CH)],
                              acc_sh.at[de_v.at[j]], sem, add=True)
             for j in range(_KA)]
            + [pltpu.async_copy(ea_v.at[pl.ds(j * _CH, _CH)],
                                acc_sh.at[do_v.at[j]], sem, add=True)
               for j in range(_KA)])
        for d in descs:
            d.wait()
        return carry

    lax.fori_loop(0, _S1, step, 0)
    plsc.subcore_barrier()
    pltpu.sync_copy(acc_sh.at[pl.ds(r0, _RPT)],
                    out_h.at[pl.ds(cid * _NPAD + r0, _RPT)])


@functools.partial(
    pl.kernel,
    out_type=jax.ShapeDtypeStruct((_NC * _NPAD, _HD), jnp.bfloat16),
    mesh=_mesh,
    scratch_types=[
        pltpu.VMEM_SHARED((_NPAD, _HD), jnp.bfloat16),
        pltpu.VMEM((_KB, _CH), jnp.int32),
        pltpu.VMEM((_KB, _CH), jnp.int32),
        pltpu.VMEM((_KB, _CH), jnp.int32),
        pltpu.VMEM((_KB, _CH), jnp.int32),
        pltpu.VMEM((_KB, _CH, _HD), jnp.bfloat16),
        pltpu.VMEM((_KB, _CH, _HD), jnp.bfloat16),
        pltpu.SemaphoreType.DMA,
        pltpu.SemaphoreType.DMA,
        pltpu.SemaphoreType.DMA,
    ],
    compiler_params=pltpu.CompilerParams(use_tc_tiling_on_sc=False),
)
def _sc_gather_scatter(table_h, src_h, dst_h, z_h, out_h, acc_sh,
                       src_a, dst_a, src_b, dst_b, rows_a, rows_b,
                       gsem, ssem, isem):
    cid = lax.axis_index("c")
    sid = lax.axis_index("s")
    r0 = sid * _RPT
    pltpu.sync_copy(z_h, acc_sh.at[pl.ds(r0, _RPT)])
    plsc.subcore_barrier()

    cbase = cid * (_E2P // _CH)
    tbase = sid * (_PW2 // _CH)

    def load_idx(g, sv, dv):
        rb = tbase + g * _KB
        da = pltpu.async_copy(src_h.at[pl.ds(cbase + rb, _KB)], sv, isem)
        db = pltpu.async_copy(dst_h.at[pl.ds(rb, _KB)], dv, isem)
        return da, db

    def fire_gathers(sv, rows):
        for j in range(_KB):
            pltpu.async_copy(table_h.at[sv.at[j]], rows.at[j], gsem)

    def drain_gathers(sv, rows):
        for j in range(_KB):
            pltpu.make_async_copy(table_h.at[sv.at[j]], rows.at[j],
                                  gsem).wait()

    def fire_scatters(dv, rows):
        for j in range(_KB):
            pltpu.async_copy(rows.at[j], acc_sh.at[dv.at[j]], ssem, add=True)

    def drain_scatters(dv, rows):
        for j in range(_KB):
            pltpu.make_async_copy(rows.at[j], acc_sh.at[dv.at[j]],
                                  ssem).wait()

    da, db = load_idx(0, src_a, dst_a)
    da.wait()
    db.wait()
    fire_gathers(src_a, rows_a)

    def step(i, carry):
        drain_gathers(src_a, rows_a)

        @pl.when(i > 0)
        def _():
            drain_scatters(dst_b, rows_b)

        da, db = load_idx(2 * i + 1, src_b, dst_b)
        fire_scatters(dst_a, rows_a)
        da.wait()
        db.wait()
        fire_gathers(src_b, rows_b)

        drain_gathers(src_b, rows_b)
        drain_scatters(dst_a, rows_a)

        @pl.when(i < _S2 // 2 - 1)
        def _():
            da2, db2 = load_idx(2 * i + 2, src_a, dst_a)
            fire_scatters(dst_b, rows_b)
            da2.wait()
            db2.wait()
            fire_gathers(src_a, rows_a)

        @pl.when(i == _S2 // 2 - 1)
        def _():
            fire_scatters(dst_b, rows_b)

        return carry

    lax.fori_loop(0, _S2 // 2, step, 0)
    drain_scatters(dst_b, rows_b)
    plsc.subcore_barrier()
    pltpu.sync_copy(acc_sh.at[pl.ds(r0, _RPT)],
                    out_h.at[pl.ds(cid * _NPAD + r0, _RPT)])


def _tc_pre_body(x_r, wg_r, acca_r, we_r, wl_r, be_r, bl_r,
                 hs_r, hsf_r, dinv_r, base2_r):
    acc = acca_r[0] + acca_r[1]
    cnt = acc[:, 16:17] + 1.0
    dinv = lax.rsqrt(cnt)
    h = jnp.dot(x_r[...], wg_r[...], preferred_element_type=jnp.float32)
    hs = h * dinv
    hsb = hs.astype(jnp.bfloat16)
    hs_r[0] = hsb[:, :_HD]
    hs_r[1] = hsb[:, _HD:]
    hsf_r[...] = hs
    dinv_r[...] = jnp.broadcast_to(dinv, (_BR, _DH))
    wl = wl_r[...]
    wl_bot = wl[_DH:, :]
    m = jnp.dot(we_r[...], wl_bot, preferred_element_type=jnp.float32)
    cvec = jnp.dot(be_r[...], wl_bot,
                   preferred_element_type=jnp.float32) + bl_r[...]
    svec = jnp.sum(m, axis=0, keepdims=True)
    base2_r[...] = (jnp.dot(acc[:, :_DE], m, preferred_element_type=jnp.float32)
                    + cnt * cvec + svec)


def _tc_pre(x, wg, acca, we, wl, be2, bl2):
    return pl.pallas_call(
        _tc_pre_body,
        grid=(_NB,),
        in_specs=[
            pl.BlockSpec((_BR, _DH), lambda i: (i, 0)),
            pl.BlockSpec((_DH, _DH), lambda i: (0, 0)),
            pl.BlockSpec((_NC, _BR, _WA), lambda i: (0, i, 0)),
            pl.BlockSpec((_DE, _DH), lambda i: (0, 0)),
            pl.BlockSpec((2 * _DH, _DH), lambda i: (0, 0)),
            pl.BlockSpec((1, _DH), lambda i: (0, 0)),
            pl.BlockSpec((1, _DH), lambda i: (0, 0)),
        ],
        out_specs=[
            pl.BlockSpec((_NC, _BR, _HD), lambda i: (0, i, 0)),
            pl.BlockSpec((_BR, _DH), lambda i: (i, 0)),
            pl.BlockSpec((_BR, _DH), lambda i: (i, 0)),
            pl.BlockSpec((_BR, _DH), lambda i: (i, 0)),
        ],
        out_shape=[
            jax.ShapeDtypeStruct((_NC, _N, _HD), jnp.bfloat16),
            jax.ShapeDtypeStruct((_N, _DH), jnp.float32),
            jax.ShapeDtypeStruct((_N, _DH), jnp.float32),
            jax.ShapeDtypeStruct((_N, _DH), jnp.float32),
        ],
    )(x, wg, acca, we, wl, be2, bl2)


def _tc_mid_body(acc1_r, hsf_r, dinv_r, wl_r, bg_r, t_r, tf_r):
    acc1 = jnp.concatenate([acc1_r[0], acc1_r[1]],
                           axis=1).astype(jnp.float32)
    pre = dinv_r[...] * (acc1 + hsf_r[...]) + bg_r[...]
    out1 = jnp.maximum(pre, 0.0)
    wl_top = wl_r[...][:_DH, :]
    t = jnp.dot(out1, wl_top, preferred_element_type=jnp.float32)
    tb = t.astype(jnp.bfloat16)
    t_r[0] = tb[:, :_HD]
    t_r[1] = tb[:, _HD:]
    tf_r[...] = t


def _tc_mid(acc1, hsf, dinvb, wl, bg2):
    return pl.pallas_call(
        _tc_mid_body,
        grid=(_NB,),
        in_specs=[
            pl.BlockSpec((_NC, _BR, _HD), lambda i: (0, i, 0)),
            pl.BlockSpec((_BR, _DH), lambda i: (i, 0)),
            pl.BlockSpec((_BR, _DH), lambda i: (i, 0)),
            pl.BlockSpec((2 * _DH, _DH), lambda i: (0, 0)),
            pl.BlockSpec((1, _DH), lambda i: (0, 0)),
        ],
        out_specs=[
            pl.BlockSpec((_NC, _BR, _HD), lambda i: (0, i, 0)),
            pl.BlockSpec((_BR, _DH), lambda i: (i, 0)),
        ],
        out_shape=[
            jax.ShapeDtypeStruct((_NC, _N, _HD), jnp.bfloat16),
            jax.ShapeDtypeStruct((_N, _DH), jnp.float32),
        ],
    )(acc1, hsf, dinvb, wl, bg2)


def _tc_post_body(acc2_r, t_r, base2_r, batch_r, wo_r, bo_r, out_r,
                  seg_acc, cnt_acc):
    i = pl.program_id(0)
    acc2 = jnp.concatenate([acc2_r[0], acc2_r[1]],
                           axis=1).astype(jnp.float32)
    out2 = jnp.maximum(acc2 + t_r[...] + base2_r[...], 0.0)
    oh = (lax.broadcasted_iota(jnp.int32, (_G, _BR), 0)
          == batch_r[0]).astype(jnp.float32)

    @pl.when(i == 0)
    def _():
        seg_acc[...] = jnp.zeros((_G, _DH), jnp.float32)
        cnt_acc[...] = jnp.zeros((_G, _DH), jnp.float32)

    seg_acc[...] += jnp.dot(oh, out2, preferred_element_type=jnp.float32)
    cnt_acc[...] += jnp.broadcast_to(
        jnp.sum(oh, axis=1, keepdims=True), (_G, _DH))
    pooled = seg_acc[...] / jnp.maximum(cnt_acc[...], 1.0)
    out_r[...] = jnp.dot(pooled, wo_r[...],
                         preferred_element_type=jnp.float32) + bo_r[...]


def _tc_post(acc2, tf, base2, batchr, wo, bo2):
    return pl.pallas_call(
        _tc_post_body,
        grid=(_NB,),
        in_specs=[
            pl.BlockSpec((_NC, _BR, _HD), lambda i: (0, i, 0)),
            pl.BlockSpec((_BR, _DH), lambda i: (i, 0)),
            pl.BlockSpec((_BR, _DH), lambda i: (i, 0)),
            pl.BlockSpec((1, 1, _BR), lambda i: (i, 0, 0)),
            pl.BlockSpec((_DH, _DH), lambda i: (0, 0)),
            pl.BlockSpec((1, _DH), lambda i: (0, 0)),
        ],
        out_specs=pl.BlockSpec((_G, _DH), lambda i: (0, 0)),
        out_shape=jax.ShapeDtypeStruct((_G, _DH), jnp.float32),
        scratch_shapes=[
            pltpu.VMEM((_G, _DH), jnp.float32),
            pltpu.VMEM((_G, _DH), jnp.float32),
        ],
    )(acc2, tf, base2, batchr, wo, bo2)


def kernel(x, edge_index, edge_attr, batch, Wg, bg, We, be, Wl, bl, Wo, bo):
    src = edge_index[0].astype(jnp.int32)
    dst = edge_index[1].astype(jnp.int32)

    pad2 = _E2P - _E2
    srcp = jnp.concatenate([src, jnp.zeros((pad2,), jnp.int32)])
    dstp = jnp.concatenate([dst, jnp.full((pad2,), _N, jnp.int32)])
    src2 = jnp.concatenate([srcp, srcp + _N]).reshape(2 * _E2P // _CH, _CH)
    dstm = dstp.reshape(_E2P // _CH, _CH)

    de = dst[0::2]
    do = dst[1::2]
    pad1 = _E1P - _EU
    dep = jnp.concatenate(
        [de, jnp.full((pad1,), _N, jnp.int32)]).reshape(_E1P // _CH, _CH)
    dop = jnp.concatenate(
        [do, jnp.full((pad1,), _N, jnp.int32)]).reshape(_E1P // _CH, _CH)
    eap = jnp.concatenate(
        [edge_attr, jnp.ones((_EU, 1), jnp.float32),
         jnp.zeros((_EU, _WA - _DE - 1), jnp.float32)], axis=1)
    eap = jnp.concatenate([eap, jnp.zeros((pad1, _WA), jnp.float32)], axis=0)

    z64 = jnp.zeros((_RPT, _HD), jnp.bfloat16)
    z32 = jnp.zeros((_RPT, _WA), jnp.float32)

    acca = _sc_edge_attr_deg(eap, dep, dop, z32)
    acca = acca.reshape(_NC, _NPAD, _WA)[:, :_N, :]

    be2 = be.reshape(1, _DH)
    bl2 = bl.reshape(1, _DH)
    bg2 = bg.reshape(1, _DH)
    bo2 = bo.reshape(1, _DH)

    hs_s, hsf, dinvb, base2 = _tc_pre(x, Wg, acca, We, Wl, be2, bl2)

    acc1 = _sc_gather_scatter(hs_s.reshape(_NC * _N, _HD), src2, dstm, z64)
    acc1 = acc1.reshape(_NC, _NPAD, _HD)[:, :_N, :]

    t_s, tf = _tc_mid(acc1, hsf, dinvb, Wl, bg2)

    acc2 = _sc_gather_scatter(t_s.reshape(_NC * _N, _HD), src2, dstm, z64)
    acc2 = acc2.reshape(_NC, _NPAD, _HD)[:, :_N, :]

    batchr = batch.astype(jnp.int32).reshape(_NB, 1, _BR)
    return _tc_post(acc2, tf, base2, batchr, Wo, bo2)
